# baseline (speedup 1.0000x reference)
import functools

import jax
import jax.numpy as jnp
from jax.experimental import pallas as pl
from jax.experimental.pallas import tpu as pltpu

_T = 384
_D = 128
_H = 256
_HH = 128
_BR = 16
_NEG = -1e30
_INV_SQRT2 = 0.7071067811865476


def _gelu_exact(x):
    return 0.5 * x * (1.0 + jax.lax.erf(x * _INV_SQRT2))


def _fused_kernel(plen_ref, tlen_ref, pair_ref,
                  W1_ref, b1_ref, g1_ref, be1_ref,
                  Wa1_ref, ba1_ref, wa2_ref, ba2_ref,
                  Wo_ref, bo_ref, g2_ref, be2_ref,
                  out_ref, m_ref, s_ref, p_ref):
    b = pl.program_id(0)
    i = pl.program_id(1)
    nb = pl.num_programs(1)

    @pl.when(i == 0)
    def _init():
        m_ref[0] = _NEG
        s_ref[0] = 0.0
        p_ref[...] = jnp.zeros_like(p_ref)

    x_in = pair_ref[...].reshape(_BR * _T, _D).astype(jnp.bfloat16)
    y = jnp.dot(x_in, W1_ref[...], preferred_element_type=jnp.float32) + b1_ref[...]
    mu = jnp.mean(y, axis=-1, keepdims=True)
    var = jnp.mean((y - mu) * (y - mu), axis=-1, keepdims=True)
    yn = (y - mu) * jax.lax.rsqrt(var + 1e-5) * g1_ref[...] + be1_ref[...]
    x = _gelu_exact(yn)

    a = jnp.tanh(jnp.dot(x.astype(jnp.bfloat16), Wa1_ref[...],
                         preferred_element_type=jnp.float32) + ba1_ref[...])
    attn = jnp.sum(a * wa2_ref[...], axis=-1, keepdims=True) + ba2_ref[0, 0]

    P = plen_ref[b]
    L = tlen_ref[b]
    k = jax.lax.broadcasted_iota(jnp.int32, (_BR * _T, 1), 0)
    g = jax.lax.shift_right_logical(
        jax.lax.shift_right_logical(k, 7) * 21846, 16)
    ri = i * _BR + g
    ci = k - g * _T
    pm_r = ri < P
    lm_r = jnp.logical_and(ri >= P, ri < L)
    pm_c = ci < P
    lm_c = jnp.logical_and(ci >= P, ci < L)
    inter = jnp.logical_or(jnp.logical_and(pm_r, lm_c),
                           jnp.logical_and(lm_r, pm_c))

    sc = jnp.where(inter, attn, _NEG)
    m_old = m_ref[0]
    m_new = jnp.maximum(m_old, jnp.max(sc))
    w = jnp.where(inter, jnp.exp(sc - m_new), 0.0)
    alpha = jnp.exp(m_old - m_new)
    m_ref[0] = m_new
    s_ref[0] = s_ref[0] * alpha + jnp.sum(w)
    p_ref[...] = p_ref[...] * alpha + jnp.sum(x * w, axis=0, keepdims=True)

    @pl.when(i == nb - 1)
    def _finish():
        pooled = p_ref[...] / jnp.maximum(s_ref[0], 1e-30)
        z = jnp.dot(pooled, Wo_ref[...], preferred_element_type=jnp.float32) + bo_ref[...]
        mu2 = jnp.mean(z, axis=-1, keepdims=True)
        var2 = jnp.mean((z - mu2) * (z - mu2), axis=-1, keepdims=True)
        zn = (z - mu2) * jax.lax.rsqrt(var2 + 1e-5) * g2_ref[...] + be2_ref[...]
        out_ref[...] = _gelu_exact(zn).reshape(out_ref.shape)


@functools.partial(jax.jit, static_argnames=())
def kernel(pair_emb, protein_length, token_length, W1, b1, g1, be1,
           Wa1, ba1, Wa2, ba2, Wo, bo, g2, be2):
    B, T, _, D = pair_emb.shape
    H = W1.shape[1]
    nb = T // _BR

    row = lambda v: v.reshape(1, -1)
    wa2_row = Wa2.reshape(1, _HH)
    W1 = W1.astype(jnp.bfloat16)
    Wa1 = Wa1.astype(jnp.bfloat16)

    grid = (B, nb)
    out = pl.pallas_call(
        _fused_kernel,
        grid=grid,
        in_specs=[
            pl.BlockSpec(memory_space=pltpu.SMEM),
            pl.BlockSpec(memory_space=pltpu.SMEM),
            pl.BlockSpec((1, _BR, T, D), lambda b, i: (b, i, 0, 0)),
            pl.BlockSpec((D, H), lambda b, i: (0, 0)),
            pl.BlockSpec((1, H), lambda b, i: (0, 0)),
            pl.BlockSpec((1, H), lambda b, i: (0, 0)),
            pl.BlockSpec((1, H), lambda b, i: (0, 0)),
            pl.BlockSpec((H, _HH), lambda b, i: (0, 0)),
            pl.BlockSpec((1, _HH), lambda b, i: (0, 0)),
            pl.BlockSpec((1, _HH), lambda b, i: (0, 0)),
            pl.BlockSpec((1, 1), lambda b, i: (0, 0)),
            pl.BlockSpec((H, H), lambda b, i: (0, 0)),
            pl.BlockSpec((1, H), lambda b, i: (0, 0)),
            pl.BlockSpec((1, H), lambda b, i: (0, 0)),
            pl.BlockSpec((1, H), lambda b, i: (0, 0)),
        ],
        out_specs=pl.BlockSpec((1, 1, H), lambda b, i: (b, 0, 0)),
        out_shape=jax.ShapeDtypeStruct((B, 1, H), jnp.float32),
        scratch_shapes=[
            pltpu.SMEM((1,), jnp.float32),
            pltpu.SMEM((1,), jnp.float32),
            pltpu.VMEM((1, H), jnp.float32),
        ],
        compiler_params=pltpu.CompilerParams(
            dimension_semantics=("arbitrary", "arbitrary"),
        ),
    )(protein_length, token_length, pair_emb,
      W1, row(b1), row(g1), row(be1),
      Wa1, row(ba1), wa2_row, ba2.reshape(1, 1),
      Wo, row(bo), row(g2), row(be2))
    return out.reshape(B, H)


# speedup vs baseline: 1.1564x; 1.1564x over previous
import functools

import jax
import jax.numpy as jnp
from jax.experimental import pallas as pl
from jax.experimental.pallas import tpu as pltpu

_T = 384
_D = 128
_H = 256
_HH = 128
_BR = 16
_NEG = -1e30
_INV_SQRT2 = 0.7071067811865476


def _gelu_exact(x):
    half = jnp.asarray(0.5, x.dtype)
    one = jnp.asarray(1.0, x.dtype)
    inv = jnp.asarray(_INV_SQRT2, x.dtype)
    return half * x * (one + jax.lax.erf(x * inv))


def _fused_kernel(plen_ref, tlen_ref, pair_ref,
                  W1_ref, b1_ref, w1m_ref, b1m_ref, g1_ref, be1_ref,
                  Wa1_ref, ba1_ref, wa2_ref, ba2_ref,
                  Wo_ref, bo_ref, g2_ref, be2_ref,
                  out_ref, m_ref, s_ref, p_ref, gf_ref, blm_ref, bpm_ref):
    b = pl.program_id(0)
    i = pl.program_id(1)
    nb = pl.num_programs(1)

    P = plen_ref[b]
    L = tlen_ref[b]

    @pl.when(i == 0)
    def _init():
        m_ref[0] = _NEG
        s_ref[0] = 0.0
        p_ref[...] = jnp.zeros_like(p_ref)
        k = jax.lax.broadcasted_iota(jnp.int32, (_BR * _T, 1), 0)
        g = jax.lax.shift_right_logical(
            jax.lax.shift_right_logical(k, 7) * 21846, 16)
        ci = k - g * _T
        gf_ref[...] = g.astype(jnp.float32)
        pm_c = ci < P
        lm_c = jnp.logical_and(ci >= P, ci < L)
        blm_ref[...] = jnp.where(lm_c, -_NEG, 0.0)
        bpm_ref[...] = jnp.where(pm_c, -_NEG, 0.0)

    x_in = pair_ref[...].reshape(_BR * _T, _D).astype(jnp.bfloat16)
    y = jnp.dot(x_in, W1_ref[...], preferred_element_type=jnp.float32) + b1_ref[...]
    mu = (jnp.dot(x_in, w1m_ref[...], preferred_element_type=jnp.float32)
          + b1m_ref[0, 0])
    e2 = jnp.mean(y * y, axis=-1, keepdims=True)
    var = jnp.maximum(e2 - mu * mu, 0.0)
    rs = jax.lax.rsqrt(var + 1e-5)

    y16 = y.astype(jnp.bfloat16)
    mu16 = mu.astype(jnp.bfloat16)
    rs16 = rs.astype(jnp.bfloat16)
    yn = (y16 - mu16) * rs16 * g1_ref[...] + be1_ref[...]
    x16 = _gelu_exact(yn)

    t = jnp.tanh(jnp.dot(x16, Wa1_ref[...],
                         preferred_element_type=jnp.float32) + ba1_ref[...])
    attn = (jnp.dot(t.astype(jnp.bfloat16), wa2_ref[...],
                    preferred_element_type=jnp.float32) + ba2_ref[0, 0])

    rp = (P - i * _BR).astype(jnp.float32)
    rl = (L - i * _BR).astype(jnp.float32)
    gf = gf_ref[...]
    is_pm = gf < rp
    is_lm = jnp.logical_and(jnp.logical_not(is_pm), gf < rl)
    bias = jnp.where(is_pm, blm_ref[...],
                     jnp.where(is_lm, bpm_ref[...], 0.0)) + _NEG
    sc = attn + bias

    m_old = m_ref[0]
    m_new = jnp.maximum(m_old, jnp.max(sc))
    w = jnp.exp(sc - m_new)
    alpha = jnp.exp(m_old - m_new)
    m_ref[0] = m_new
    s_ref[0] = s_ref[0] * alpha + jnp.sum(w)
    pc = jax.lax.dot_general(w.astype(jnp.bfloat16), x16,
                             dimension_numbers=(((0,), (0,)), ((), ())),
                             preferred_element_type=jnp.float32)
    p_ref[...] = p_ref[...] * alpha + pc

    @pl.when(i == nb - 1)
    def _finish():
        pooled = p_ref[...] / jnp.maximum(s_ref[0], 1e-30)
        z = jnp.dot(pooled, Wo_ref[...], preferred_element_type=jnp.float32) + bo_ref[...]
        mu2 = jnp.mean(z, axis=-1, keepdims=True)
        var2 = jnp.mean((z - mu2) * (z - mu2), axis=-1, keepdims=True)
        zn = (z - mu2) * jax.lax.rsqrt(var2 + 1e-5) * g2_ref[...] + be2_ref[...]
        out_ref[...] = _gelu_exact(zn).reshape(out_ref.shape)


@functools.partial(jax.jit, static_argnames=())
def kernel(pair_emb, protein_length, token_length, W1, b1, g1, be1,
           Wa1, ba1, Wa2, ba2, Wo, bo, g2, be2):
    B, T, _, D = pair_emb.shape
    H = W1.shape[1]
    nb = T // _BR

    row = lambda v: v.reshape(1, -1)
    w1m = jnp.mean(W1, axis=1, keepdims=True).astype(jnp.bfloat16)
    b1m = jnp.mean(b1).reshape(1, 1)
    W1_16 = W1.astype(jnp.bfloat16)
    Wa1_16 = Wa1.astype(jnp.bfloat16)
    wa2_16 = Wa2.astype(jnp.bfloat16)
    g1_16 = row(g1).astype(jnp.bfloat16)
    be1_16 = row(be1).astype(jnp.bfloat16)

    grid = (B, nb)
    const = lambda b, i: (0, 0)
    out = pl.pallas_call(
        _fused_kernel,
        grid=grid,
        in_specs=[
            pl.BlockSpec(memory_space=pltpu.SMEM),
            pl.BlockSpec(memory_space=pltpu.SMEM),
            pl.BlockSpec((1, _BR, T, D), lambda b, i: (b, i, 0, 0)),
            pl.BlockSpec((D, H), const),
            pl.BlockSpec((1, H), const),
            pl.BlockSpec((D, 1), const),
            pl.BlockSpec((1, 1), const),
            pl.BlockSpec((1, H), const),
            pl.BlockSpec((1, H), const),
            pl.BlockSpec((H, _HH), const),
            pl.BlockSpec((1, _HH), const),
            pl.BlockSpec((_HH, 1), const),
            pl.BlockSpec((1, 1), const),
            pl.BlockSpec((H, H), const),
            pl.BlockSpec((1, H), const),
            pl.BlockSpec((1, H), const),
            pl.BlockSpec((1, H), const),
        ],
        out_specs=pl.BlockSpec((1, 1, H), lambda b, i: (b, 0, 0)),
        out_shape=jax.ShapeDtypeStruct((B, 1, H), jnp.float32),
        scratch_shapes=[
            pltpu.SMEM((1,), jnp.float32),
            pltpu.SMEM((1,), jnp.float32),
            pltpu.VMEM((1, H), jnp.float32),
            pltpu.VMEM((_BR * _T, 1), jnp.float32),
            pltpu.VMEM((_BR * _T, 1), jnp.float32),
            pltpu.VMEM((_BR * _T, 1), jnp.float32),
        ],
        compiler_params=pltpu.CompilerParams(
            dimension_semantics=("arbitrary", "arbitrary"),
        ),
    )(protein_length, token_length, pair_emb,
      W1_16, row(b1), w1m, b1m, g1_16, be1_16,
      Wa1_16, row(ba1), wa2_16, ba2.reshape(1, 1),
      Wo, row(bo), row(g2), row(be2))
    return out.reshape(B, H)


# speedup vs baseline: 1.2835x; 1.1099x over previous
import functools

import jax
import jax.numpy as jnp
from jax.experimental import pallas as pl
from jax.experimental.pallas import tpu as pltpu

_T = 384
_D = 128
_H = 256
_HH = 128
_BR = 16
_N = _BR * _T
_NEG = -1e30
_INV_SQRT2 = 0.7071067811865476


def _gelu_exact(x):
    half = jnp.asarray(0.5, x.dtype)
    one = jnp.asarray(1.0, x.dtype)
    inv = jnp.asarray(_INV_SQRT2, x.dtype)
    return half * x * (one + jax.lax.erf(x * inv))


_NT = (((1,), (1,)), ((), ()))


def _fused_kernel(plen_ref, tlen_ref, pair_ref,
                  W1_ref, b1_ref, w1m_ref, b1m_ref, g1_ref, be1_ref,
                  Wa1_ref, ba1_ref, wa2_ref, ba2_ref,
                  Wo_ref, bo_ref, g2_ref, be2_ref,
                  out_ref, m_ref, s_ref, p_ref, gf_ref, blm_ref, bpm_ref):
    b = pl.program_id(0)
    i = pl.program_id(1)
    nb = pl.num_programs(1)

    P = plen_ref[b]
    L = tlen_ref[b]

    @pl.when(i == 0)
    def _init():
        m_ref[0] = _NEG
        s_ref[0] = 0.0
        p_ref[...] = jnp.zeros_like(p_ref)
        k = jax.lax.broadcasted_iota(jnp.int32, (1, _N), 1)
        g = jax.lax.shift_right_logical(
            jax.lax.shift_right_logical(k, 7) * 21846, 16)
        ci = k - g * _T
        gf_ref[...] = g.astype(jnp.float32)
        pm_c = ci < P
        lm_c = jnp.logical_and(ci >= P, ci < L)
        blm_ref[...] = jnp.where(lm_c, -_NEG, 0.0)
        bpm_ref[...] = jnp.where(pm_c, -_NEG, 0.0)

    x_in = pair_ref[...].reshape(_N, _D).astype(jnp.bfloat16)
    y = jnp.dot(x_in, W1_ref[...], preferred_element_type=jnp.float32) + b1_ref[...]
    mu = (jnp.dot(x_in, w1m_ref[...], preferred_element_type=jnp.float32)
          + b1m_ref[0, 0])
    y16 = y.astype(jnp.bfloat16)
    ones_h = jnp.ones((_H, 1), jnp.bfloat16)
    e2 = jax.lax.dot_general(y16 * y16, ones_h, (((1,), (0,)), ((), ())),
                             preferred_element_type=jnp.float32) * (1.0 / _H)
    var = jnp.maximum(e2 - mu * mu, 0.0)
    rs = jax.lax.rsqrt(var + 1e-5)

    mu16 = mu.astype(jnp.bfloat16)
    rs16 = rs.astype(jnp.bfloat16)
    yn = (y16 - mu16) * rs16 * g1_ref[...] + be1_ref[...]
    x16 = _gelu_exact(yn)

    t = jnp.dot(x16, Wa1_ref[...], preferred_element_type=jnp.float32) + ba1_ref[...]
    t16 = jnp.tanh(t.astype(jnp.bfloat16))
    attn = (jax.lax.dot_general(wa2_ref[...], t16, _NT,
                                preferred_element_type=jnp.float32)
            + ba2_ref[0, 0])

    rp = (P - i * _BR).astype(jnp.float32)
    rl = (L - i * _BR).astype(jnp.float32)
    gf = gf_ref[...]
    is_pm = gf < rp
    is_lm = jnp.logical_and(jnp.logical_not(is_pm), gf < rl)
    bias = jnp.where(is_pm, blm_ref[...],
                     jnp.where(is_lm, bpm_ref[...], 0.0)) + _NEG
    sc = attn + bias

    m_old = m_ref[0]
    m_new = jnp.maximum(m_old, jnp.max(sc))
    w = jnp.exp(sc - m_new)
    alpha = jnp.exp(m_old - m_new)
    m_ref[0] = m_new
    s_ref[0] = s_ref[0] * alpha + jnp.sum(w)
    pc = jnp.dot(w.astype(jnp.bfloat16), x16,
                 preferred_element_type=jnp.float32)
    p_ref[...] = p_ref[...] * alpha + pc

    @pl.when(i == nb - 1)
    def _finish():
        pooled = p_ref[...] / jnp.maximum(s_ref[0], 1e-30)
        z = jnp.dot(pooled, Wo_ref[...], preferred_element_type=jnp.float32) + bo_ref[...]
        mu2 = jnp.mean(z, axis=-1, keepdims=True)
        var2 = jnp.mean((z - mu2) * (z - mu2), axis=-1, keepdims=True)
        zn = (z - mu2) * jax.lax.rsqrt(var2 + 1e-5) * g2_ref[...] + be2_ref[...]
        out_ref[...] = _gelu_exact(zn).reshape(out_ref.shape)


@functools.partial(jax.jit, static_argnames=())
def kernel(pair_emb, protein_length, token_length, W1, b1, g1, be1,
           Wa1, ba1, Wa2, ba2, Wo, bo, g2, be2):
    B, T, _, D = pair_emb.shape
    H = W1.shape[1]
    nb = T // _BR

    row = lambda v: v.reshape(1, -1)
    w1m = jnp.mean(W1, axis=1, keepdims=True).astype(jnp.bfloat16)
    b1m = jnp.mean(b1).reshape(1, 1)
    W1_16 = W1.astype(jnp.bfloat16)
    Wa1_16 = Wa1.astype(jnp.bfloat16)
    wa2_row = Wa2.reshape(1, _HH).astype(jnp.bfloat16)
    g1_16 = row(g1).astype(jnp.bfloat16)
    be1_16 = row(be1).astype(jnp.bfloat16)
    ba1_16 = row(ba1)

    grid = (B, nb)
    const = lambda b, i: (0, 0)
    out = pl.pallas_call(
        _fused_kernel,
        grid=grid,
        in_specs=[
            pl.BlockSpec(memory_space=pltpu.SMEM),
            pl.BlockSpec(memory_space=pltpu.SMEM),
            pl.BlockSpec((1, _BR, T, D), lambda b, i: (b, i, 0, 0)),
            pl.BlockSpec((D, H), const),
            pl.BlockSpec((1, H), const),
            pl.BlockSpec((D, 1), const),
            pl.BlockSpec((1, 1), const),
            pl.BlockSpec((1, H), const),
            pl.BlockSpec((1, H), const),
            pl.BlockSpec((H, _HH), const),
            pl.BlockSpec((1, _HH), const),
            pl.BlockSpec((1, _HH), const),
            pl.BlockSpec((1, 1), const),
            pl.BlockSpec((H, H), const),
            pl.BlockSpec((1, H), const),
            pl.BlockSpec((1, H), const),
            pl.BlockSpec((1, H), const),
        ],
        out_specs=pl.BlockSpec((1, 1, H), lambda b, i: (b, 0, 0)),
        out_shape=jax.ShapeDtypeStruct((B, 1, H), jnp.float32),
        scratch_shapes=[
            pltpu.SMEM((1,), jnp.float32),
            pltpu.SMEM((1,), jnp.float32),
            pltpu.VMEM((1, _H), jnp.float32),
            pltpu.VMEM((1, _N), jnp.float32),
            pltpu.VMEM((1, _N), jnp.float32),
            pltpu.VMEM((1, _N), jnp.float32),
        ],
        compiler_params=pltpu.CompilerParams(
            dimension_semantics=("arbitrary", "arbitrary"),
        ),
    )(protein_length, token_length, pair_emb,
      W1_16, row(b1), w1m, b1m, g1_16, be1_16,
      Wa1_16, ba1_16, wa2_row, ba2.reshape(1, 1),
      Wo, row(bo), row(g2), row(be2))
    return out.reshape(B, H)


# speedup vs baseline: 1.3140x; 1.0238x over previous
import functools

import jax
import jax.numpy as jnp
from jax.experimental import pallas as pl
from jax.experimental.pallas import tpu as pltpu

_T = 384
_D = 128
_H = 256
_HH = 128
_BR = 32
_N = _BR * _T
_NEG = -1e30
_INV_SQRT2 = 0.7071067811865476


def _gelu_exact(x):
    half = jnp.asarray(0.5, x.dtype)
    one = jnp.asarray(1.0, x.dtype)
    inv = jnp.asarray(_INV_SQRT2, x.dtype)
    return half * x * (one + jax.lax.erf(x * inv))


_NT = (((1,), (1,)), ((), ()))


def _fused_kernel(plen_ref, tlen_ref, pair_ref,
                  W1_ref, b1_ref, w1m_ref, b1m_ref, g1_ref, be1_ref,
                  Wa1_ref, ba1_ref, wa2_ref, ba2_ref,
                  Wo_ref, bo_ref, g2_ref, be2_ref,
                  out_ref, m_ref, s_ref, p_ref, gf_ref, blm_ref, bpm_ref):
    b = pl.program_id(0)
    i = pl.program_id(1)
    nb = pl.num_programs(1)

    P = plen_ref[b]
    L = tlen_ref[b]

    @pl.when(i == 0)
    def _init():
        m_ref[0] = _NEG
        s_ref[0] = 0.0
        p_ref[...] = jnp.zeros_like(p_ref)
        k = jax.lax.broadcasted_iota(jnp.int32, (1, _N), 1)
        g = jax.lax.shift_right_logical(
            jax.lax.shift_right_logical(k, 7) * 21846, 16)
        ci = k - g * _T
        gf_ref[...] = g.astype(jnp.float32)
        pm_c = ci < P
        lm_c = jnp.logical_and(ci >= P, ci < L)
        blm_ref[...] = jnp.where(lm_c, -_NEG, 0.0)
        bpm_ref[...] = jnp.where(pm_c, -_NEG, 0.0)

    x_in = pair_ref[...].reshape(_N, _D).astype(jnp.bfloat16)
    y = jnp.dot(x_in, W1_ref[...], preferred_element_type=jnp.float32) + b1_ref[...]
    mu = (jnp.dot(x_in, w1m_ref[...], preferred_element_type=jnp.float32)
          + b1m_ref[0, 0])
    y16 = y.astype(jnp.bfloat16)
    ones_h = jnp.ones((_H, 1), jnp.bfloat16)
    e2 = jax.lax.dot_general(y16 * y16, ones_h, (((1,), (0,)), ((), ())),
                             preferred_element_type=jnp.float32) * (1.0 / _H)
    var = jnp.maximum(e2 - mu * mu, 0.0)
    rs = jax.lax.rsqrt(var + 1e-5)

    mu16 = mu.astype(jnp.bfloat16)
    rs16 = rs.astype(jnp.bfloat16)
    yn = (y16 - mu16) * rs16 * g1_ref[...] + be1_ref[...]
    x16 = _gelu_exact(yn)

    t = jnp.dot(x16, Wa1_ref[...], preferred_element_type=jnp.float32) + ba1_ref[...]
    t16 = jnp.tanh(t.astype(jnp.bfloat16))
    attn = (jax.lax.dot_general(wa2_ref[...], t16, _NT,
                                preferred_element_type=jnp.float32)
            + ba2_ref[0, 0])

    rp = (P - i * _BR).astype(jnp.float32)
    rl = (L - i * _BR).astype(jnp.float32)
    gf = gf_ref[...]
    is_pm = gf < rp
    is_lm = jnp.logical_and(jnp.logical_not(is_pm), gf < rl)
    bias = jnp.where(is_pm, blm_ref[...],
                     jnp.where(is_lm, bpm_ref[...], 0.0)) + _NEG
    sc = attn + bias

    m_old = m_ref[0]
    m_new = jnp.maximum(m_old, jnp.max(sc))
    w = jnp.exp(sc - m_new)
    alpha = jnp.exp(m_old - m_new)
    m_ref[0] = m_new
    s_ref[0] = s_ref[0] * alpha + jnp.sum(w)
    pc = jnp.dot(w.astype(jnp.bfloat16), x16,
                 preferred_element_type=jnp.float32)
    p_ref[...] = p_ref[...] * alpha + pc

    @pl.when(i == nb - 1)
    def _finish():
        pooled = p_ref[...] / jnp.maximum(s_ref[0], 1e-30)
        z = jnp.dot(pooled, Wo_ref[...], preferred_element_type=jnp.float32) + bo_ref[...]
        mu2 = jnp.mean(z, axis=-1, keepdims=True)
        var2 = jnp.mean((z - mu2) * (z - mu2), axis=-1, keepdims=True)
        zn = (z - mu2) * jax.lax.rsqrt(var2 + 1e-5) * g2_ref[...] + be2_ref[...]
        out_ref[...] = _gelu_exact(zn).reshape(out_ref.shape)


@functools.partial(jax.jit, static_argnames=())
def kernel(pair_emb, protein_length, token_length, W1, b1, g1, be1,
           Wa1, ba1, Wa2, ba2, Wo, bo, g2, be2):
    B, T, _, D = pair_emb.shape
    H = W1.shape[1]
    nb = T // _BR

    row = lambda v: v.reshape(1, -1)
    w1m = jnp.mean(W1, axis=1, keepdims=True).astype(jnp.bfloat16)
    b1m = jnp.mean(b1).reshape(1, 1)
    W1_16 = W1.astype(jnp.bfloat16)
    Wa1_16 = Wa1.astype(jnp.bfloat16)
    wa2_row = Wa2.reshape(1, _HH).astype(jnp.bfloat16)
    g1_16 = row(g1).astype(jnp.bfloat16)
    be1_16 = row(be1).astype(jnp.bfloat16)
    ba1_16 = row(ba1)

    grid = (B, nb)
    const = lambda b, i: (0, 0)
    out = pl.pallas_call(
        _fused_kernel,
        grid=grid,
        in_specs=[
            pl.BlockSpec(memory_space=pltpu.SMEM),
            pl.BlockSpec(memory_space=pltpu.SMEM),
            pl.BlockSpec((1, _BR, T, D), lambda b, i: (b, i, 0, 0)),
            pl.BlockSpec((D, H), const),
            pl.BlockSpec((1, H), const),
            pl.BlockSpec((D, 1), const),
            pl.BlockSpec((1, 1), const),
            pl.BlockSpec((1, H), const),
            pl.BlockSpec((1, H), const),
            pl.BlockSpec((H, _HH), const),
            pl.BlockSpec((1, _HH), const),
            pl.BlockSpec((1, _HH), const),
            pl.BlockSpec((1, 1), const),
            pl.BlockSpec((H, H), const),
            pl.BlockSpec((1, H), const),
            pl.BlockSpec((1, H), const),
            pl.BlockSpec((1, H), const),
        ],
        out_specs=pl.BlockSpec((1, 1, H), lambda b, i: (b, 0, 0)),
        out_shape=jax.ShapeDtypeStruct((B, 1, H), jnp.float32),
        scratch_shapes=[
            pltpu.SMEM((1,), jnp.float32),
            pltpu.SMEM((1,), jnp.float32),
            pltpu.VMEM((1, _H), jnp.float32),
            pltpu.VMEM((1, _N), jnp.float32),
            pltpu.VMEM((1, _N), jnp.float32),
            pltpu.VMEM((1, _N), jnp.float32),
        ],
        compiler_params=pltpu.CompilerParams(
            dimension_semantics=("arbitrary", "arbitrary"),
        ),
    )(protein_length, token_length, pair_emb,
      W1_16, row(b1), w1m, b1m, g1_16, be1_16,
      Wa1_16, ba1_16, wa2_row, ba2.reshape(1, 1),
      Wo, row(bo), row(g2), row(be2))
    return out.reshape(B, H)


# speedup vs baseline: 1.5623x; 1.1889x over previous
import functools

import jax
import jax.numpy as jnp
from jax.experimental import pallas as pl
from jax.experimental.pallas import tpu as pltpu

_T = 384
_D = 128
_H = 256
_HH = 128
_BR = 32
_N = _BR * _T
_NEG = -1e30
_INV_SQRT2 = 0.7071067811865476


def _gelu_exact(x):
    half = jnp.asarray(0.5, x.dtype)
    one = jnp.asarray(1.0, x.dtype)
    inv = jnp.asarray(_INV_SQRT2, x.dtype)
    return half * x * (one + jax.lax.erf(x * inv))


_NT = (((1,), (1,)), ((), ()))


def _fused_kernel(plen_ref, tlen_ref, pair_ref,
                  W1_ref, b1_ref, g1_ref, be1_ref,
                  Wa1_ref, ba1_ref, wa2_ref, ba2_ref,
                  Wo_ref, bo_ref, g2_ref, be2_ref,
                  out_ref, m_ref, s_ref, p_ref, gf_ref, blm_ref, bpm_ref):
    b = pl.program_id(0)
    i = pl.program_id(1)
    nb = pl.num_programs(1)

    P = plen_ref[b]
    L = tlen_ref[b]

    @pl.when(i == 0)
    def _init():
        m_ref[0] = _NEG
        s_ref[0] = 0.0
        p_ref[...] = jnp.zeros_like(p_ref)
        k = jax.lax.broadcasted_iota(jnp.int32, (1, _N), 1)
        g = jax.lax.shift_right_logical(
            jax.lax.shift_right_logical(k, 7) * 21846, 16)
        ci = k - g * _T
        gf_ref[...] = g.astype(jnp.float32)
        pm_c = ci < P
        lm_c = jnp.logical_and(ci >= P, ci < L)
        blm_ref[...] = jnp.where(lm_c, -_NEG, 0.0)
        bpm_ref[...] = jnp.where(pm_c, -_NEG, 0.0)

    x_in = pair_ref[...].reshape(_N, _D).astype(jnp.bfloat16)
    y16 = (jnp.dot(x_in, W1_ref[...], preferred_element_type=jnp.float32)
           .astype(jnp.bfloat16) + b1_ref[...])
    invh = jnp.full((_H, 1), 1.0 / _H, jnp.bfloat16)
    var = jax.lax.dot_general(y16 * y16, invh, (((1,), (0,)), ((), ())),
                              preferred_element_type=jnp.float32)
    rs16 = jax.lax.rsqrt(var + 1e-5).astype(jnp.bfloat16)

    yn = y16 * rs16 * g1_ref[...] + be1_ref[...]
    x16 = _gelu_exact(yn)

    t = jnp.dot(x16, Wa1_ref[...], preferred_element_type=jnp.float32) + ba1_ref[...]
    t16 = jnp.tanh(t.astype(jnp.bfloat16))
    attn = (jax.lax.dot_general(wa2_ref[...], t16, _NT,
                                preferred_element_type=jnp.float32)
            + ba2_ref[0, 0])

    rp = (P - i * _BR).astype(jnp.float32)
    rl = (L - i * _BR).astype(jnp.float32)
    gf = gf_ref[...]
    is_pm = gf < rp
    is_lm = jnp.logical_and(jnp.logical_not(is_pm), gf < rl)
    bias = jnp.where(is_pm, blm_ref[...],
                     jnp.where(is_lm, bpm_ref[...], 0.0)) + _NEG
    sc = attn + bias

    m_old = m_ref[0]
    m_new = jnp.maximum(m_old, jnp.max(sc))
    w = jnp.exp(sc - m_new)
    alpha = jnp.exp(m_old - m_new)
    m_ref[0] = m_new
    s_ref[0] = s_ref[0] * alpha + jnp.sum(w)
    pc = jnp.dot(w.astype(jnp.bfloat16), x16,
                 preferred_element_type=jnp.float32)
    p_ref[...] = p_ref[...] * alpha + pc

    @pl.when(i == nb - 1)
    def _finish():
        pooled = p_ref[...] / jnp.maximum(s_ref[0], 1e-30)
        z = jnp.dot(pooled, Wo_ref[...], preferred_element_type=jnp.float32) + bo_ref[...]
        mu2 = jnp.mean(z, axis=-1, keepdims=True)
        var2 = jnp.mean((z - mu2) * (z - mu2), axis=-1, keepdims=True)
        zn = (z - mu2) * jax.lax.rsqrt(var2 + 1e-5) * g2_ref[...] + be2_ref[...]
        out_ref[...] = _gelu_exact(zn).reshape(out_ref.shape)


@functools.partial(jax.jit, static_argnames=())
def kernel(pair_emb, protein_length, token_length, W1, b1, g1, be1,
           Wa1, ba1, Wa2, ba2, Wo, bo, g2, be2):
    B, T, _, D = pair_emb.shape
    H = W1.shape[1]
    nb = T // _BR

    row = lambda v: v.reshape(1, -1)
    W1_16 = (W1 - jnp.mean(W1, axis=1, keepdims=True)).astype(jnp.bfloat16)
    b1c_16 = row(b1 - jnp.mean(b1)).astype(jnp.bfloat16)
    Wa1_16 = Wa1.astype(jnp.bfloat16)
    wa2_row = Wa2.reshape(1, _HH).astype(jnp.bfloat16)
    g1_16 = row(g1).astype(jnp.bfloat16)
    be1_16 = row(be1).astype(jnp.bfloat16)
    ba1_16 = row(ba1)

    grid = (B, nb)
    const = lambda b, i: (0, 0)
    out = pl.pallas_call(
        _fused_kernel,
        grid=grid,
        in_specs=[
            pl.BlockSpec(memory_space=pltpu.SMEM),
            pl.BlockSpec(memory_space=pltpu.SMEM),
            pl.BlockSpec((1, _BR, T, D), lambda b, i: (b, i, 0, 0)),
            pl.BlockSpec((D, H), const),
            pl.BlockSpec((1, H), const),
            pl.BlockSpec((1, H), const),
            pl.BlockSpec((1, H), const),
            pl.BlockSpec((H, _HH), const),
            pl.BlockSpec((1, _HH), const),
            pl.BlockSpec((1, _HH), const),
            pl.BlockSpec((1, 1), const),
            pl.BlockSpec((H, H), const),
            pl.BlockSpec((1, H), const),
            pl.BlockSpec((1, H), const),
            pl.BlockSpec((1, H), const),
        ],
        out_specs=pl.BlockSpec((1, 1, H), lambda b, i: (b, 0, 0)),
        out_shape=jax.ShapeDtypeStruct((B, 1, H), jnp.float32),
        scratch_shapes=[
            pltpu.SMEM((1,), jnp.float32),
            pltpu.SMEM((1,), jnp.float32),
            pltpu.VMEM((1, _H), jnp.float32),
            pltpu.VMEM((1, _N), jnp.float32),
            pltpu.VMEM((1, _N), jnp.float32),
            pltpu.VMEM((1, _N), jnp.float32),
        ],
        compiler_params=pltpu.CompilerParams(
            dimension_semantics=("arbitrary", "arbitrary"),
        ),
    )(protein_length, token_length, pair_emb,
      W1_16, b1c_16, g1_16, be1_16,
      Wa1_16, ba1_16, wa2_row, ba2.reshape(1, 1),
      Wo, row(bo), row(g2), row(be2))
    return out.reshape(B, H)


# speedup vs baseline: 1.5771x; 1.0095x over previous
import functools

import jax
import jax.numpy as jnp
from jax.experimental import pallas as pl
from jax.experimental.pallas import tpu as pltpu

_T = 384
_D = 128
_H = 256
_HH = 128
_BR = 48
_N = _BR * _T
_NEG = -1e30
_INV_SQRT2 = 0.7071067811865476


def _gelu_exact(x):
    half = jnp.asarray(0.5, x.dtype)
    one = jnp.asarray(1.0, x.dtype)
    inv = jnp.asarray(_INV_SQRT2, x.dtype)
    return half * x * (one + jax.lax.erf(x * inv))


_NT = (((1,), (1,)), ((), ()))


def _fused_kernel(plen_ref, tlen_ref, pair_ref,
                  W1_ref, b1_ref, g1_ref, be1_ref,
                  Wa1_ref, ba1_ref, wa2_ref, ba2_ref,
                  Wo_ref, bo_ref, g2_ref, be2_ref,
                  out_ref, m_ref, s_ref, p_ref, gf_ref, blm_ref, bpm_ref):
    b = pl.program_id(0)
    i = pl.program_id(1)
    nb = pl.num_programs(1)

    P = plen_ref[b]
    L = tlen_ref[b]

    @pl.when(i == 0)
    def _init():
        m_ref[0] = _NEG
        s_ref[0] = 0.0
        p_ref[...] = jnp.zeros_like(p_ref)
        k = jax.lax.broadcasted_iota(jnp.int32, (1, _N), 1)
        g = jax.lax.shift_right_logical(
            jax.lax.shift_right_logical(k, 7) * 21846, 16)
        ci = k - g * _T
        gf_ref[...] = g.astype(jnp.float32)
        pm_c = ci < P
        lm_c = jnp.logical_and(ci >= P, ci < L)
        blm_ref[...] = jnp.where(lm_c, -_NEG, 0.0)
        bpm_ref[...] = jnp.where(pm_c, -_NEG, 0.0)

    x_in = pair_ref[...].reshape(_N, _D).astype(jnp.bfloat16)
    y16 = (jnp.dot(x_in, W1_ref[...], preferred_element_type=jnp.float32)
           .astype(jnp.bfloat16) + b1_ref[...])
    invh = jnp.full((_H, 1), 1.0 / _H, jnp.bfloat16)
    var = jax.lax.dot_general(y16 * y16, invh, (((1,), (0,)), ((), ())),
                              preferred_element_type=jnp.float32)
    rs16 = jax.lax.rsqrt(var + 1e-5).astype(jnp.bfloat16)

    yn = y16 * rs16 * g1_ref[...] + be1_ref[...]
    x16 = _gelu_exact(yn)

    t16 = jnp.tanh((jnp.dot(x16, Wa1_ref[...],
                            preferred_element_type=jnp.float32)
                    .astype(jnp.bfloat16)) + ba1_ref[...])
    attn = (jax.lax.dot_general(wa2_ref[...], t16, _NT,
                                preferred_element_type=jnp.float32)
            + ba2_ref[0, 0])

    rp = (P - i * _BR).astype(jnp.float32)
    rl = (L - i * _BR).astype(jnp.float32)
    gf = gf_ref[...]
    is_pm = gf < rp
    is_lm = jnp.logical_and(jnp.logical_not(is_pm), gf < rl)
    bias = jnp.where(is_pm, blm_ref[...],
                     jnp.where(is_lm, bpm_ref[...], 0.0)) + _NEG
    sc = attn + bias

    m_old = m_ref[0]
    m_new = jnp.maximum(m_old, jnp.max(sc))
    w = jnp.exp(sc - m_new)
    alpha = jnp.exp(m_old - m_new)
    m_ref[0] = m_new
    s_ref[0] = s_ref[0] * alpha + jnp.sum(w)
    pc = jnp.dot(w.astype(jnp.bfloat16), x16,
                 preferred_element_type=jnp.float32)
    p_ref[...] = p_ref[...] * alpha + pc

    @pl.when(i == nb - 1)
    def _finish():
        pooled = p_ref[...] / jnp.maximum(s_ref[0], 1e-30)
        z = jnp.dot(pooled, Wo_ref[...], preferred_element_type=jnp.float32) + bo_ref[...]
        mu2 = jnp.mean(z, axis=-1, keepdims=True)
        var2 = jnp.mean((z - mu2) * (z - mu2), axis=-1, keepdims=True)
        zn = (z - mu2) * jax.lax.rsqrt(var2 + 1e-5) * g2_ref[...] + be2_ref[...]
        out_ref[...] = _gelu_exact(zn).reshape(out_ref.shape)


@functools.partial(jax.jit, static_argnames=())
def kernel(pair_emb, protein_length, token_length, W1, b1, g1, be1,
           Wa1, ba1, Wa2, ba2, Wo, bo, g2, be2):
    B, T, _, D = pair_emb.shape
    H = W1.shape[1]
    nb = T // _BR

    row = lambda v: v.reshape(1, -1)
    W1_16 = (W1 - jnp.mean(W1, axis=1, keepdims=True)).astype(jnp.bfloat16)
    b1c_16 = row(b1 - jnp.mean(b1)).astype(jnp.bfloat16)
    Wa1_16 = Wa1.astype(jnp.bfloat16)
    wa2_row = Wa2.reshape(1, _HH).astype(jnp.bfloat16)
    g1_16 = row(g1).astype(jnp.bfloat16)
    be1_16 = row(be1).astype(jnp.bfloat16)
    ba1_16 = row(ba1).astype(jnp.bfloat16)

    grid = (B, nb)
    const = lambda b, i: (0, 0)
    out = pl.pallas_call(
        _fused_kernel,
        grid=grid,
        in_specs=[
            pl.BlockSpec(memory_space=pltpu.SMEM),
            pl.BlockSpec(memory_space=pltpu.SMEM),
            pl.BlockSpec((1, _BR, T, D), lambda b, i: (b, i, 0, 0)),
            pl.BlockSpec((D, H), const),
            pl.BlockSpec((1, H), const),
            pl.BlockSpec((1, H), const),
            pl.BlockSpec((1, H), const),
            pl.BlockSpec((H, _HH), const),
            pl.BlockSpec((1, _HH), const),
            pl.BlockSpec((1, _HH), const),
            pl.BlockSpec((1, 1), const),
            pl.BlockSpec((H, H), const),
            pl.BlockSpec((1, H), const),
            pl.BlockSpec((1, H), const),
            pl.BlockSpec((1, H), const),
        ],
        out_specs=pl.BlockSpec((1, 1, H), lambda b, i: (b, 0, 0)),
        out_shape=jax.ShapeDtypeStruct((B, 1, H), jnp.float32),
        scratch_shapes=[
            pltpu.SMEM((1,), jnp.float32),
            pltpu.SMEM((1,), jnp.float32),
            pltpu.VMEM((1, _H), jnp.float32),
            pltpu.VMEM((1, _N), jnp.float32),
            pltpu.VMEM((1, _N), jnp.float32),
            pltpu.VMEM((1, _N), jnp.float32),
        ],
        compiler_params=pltpu.CompilerParams(
            dimension_semantics=("arbitrary", "arbitrary"),
        ),
    )(protein_length, token_length, pair_emb,
      W1_16, b1c_16, g1_16, be1_16,
      Wa1_16, ba1_16, wa2_row, ba2.reshape(1, 1),
      Wo, row(bo), row(g2), row(be2))
    return out.reshape(B, H)
